# Initial kernel scaffold; baseline (speedup 1.0000x reference)
import jax
import jax.numpy as jnp
from jax.experimental import pallas as pl


def kernel(x, edge_index, W, att_src, att_dst):
    raise NotImplementedError("write your pallas kernel here")


# speedup vs baseline: 3.2788x; 3.2788x over previous
import functools

import jax
import jax.numpy as jnp
from jax import lax
from jax.experimental import pallas as pl
from jax.experimental.pallas import tpu as pltpu
from jax.experimental.pallas import tpu_sc as plsc

N = 10000
NP = N // 2
E = 160000
F = 256
FH = 128
NC = 2
NS = 16
RB = 2000
EPT = E // NS
C2 = 80
NCH = EPT // C2
E3 = E // 25


def _tc_body(x_ref, w_ref, as_ref, ad_ref, h0_ref, h1_ref,
             as5_ref, ad5_ref, m_ref, acc_ref):
    i = pl.program_id(0)
    h = lax.dot_general(x_ref[...], w_ref[...], (((1,), (1,)), ((), ())),
                        preferred_element_type=jnp.float32)
    h0_ref[...] = h[:, :FH]
    h1_ref[...] = h[:, FH:]
    a_s = lax.dot_general(as_ref[...], h, (((1,), (1,)), ((), ())),
                          preferred_element_type=jnp.float32)
    a_d = lax.dot_general(ad_ref[...], h, (((1,), (1,)), ((), ())),
                          preferred_element_type=jnp.float32)
    as5_ref[0, 0, :] = a_s[0]
    ad5_ref[0, 0, :] = a_d[0]

    @pl.when(i == 0)
    def _():
        acc_ref[0] = -jnp.inf
        acc_ref[1] = -jnp.inf

    acc_ref[0] = jnp.maximum(acc_ref[0], jnp.max(a_s))
    acc_ref[1] = jnp.maximum(acc_ref[1], jnp.max(a_d))

    @pl.when(i == pl.num_programs(0) - 1)
    def _():
        m_ref[...] = jnp.full((1, 128), acc_ref[0] + acc_ref[1], jnp.float32)


_tc_call = pl.pallas_call(
    _tc_body,
    grid=(N // RB,),
    in_specs=[
        pl.BlockSpec((RB, F), lambda i: (i, 0)),
        pl.BlockSpec((F, F), lambda i: (0, 0)),
        pl.BlockSpec((1, F), lambda i: (0, 0)),
        pl.BlockSpec((1, F), lambda i: (0, 0)),
    ],
    out_specs=[
        pl.BlockSpec((RB, FH), lambda i: (i, 0)),
        pl.BlockSpec((RB, FH), lambda i: (i, 0)),
        pl.BlockSpec((1, 1, RB), lambda i: (i, 0, 0)),
        pl.BlockSpec((1, 1, RB), lambda i: (i, 0, 0)),
        pl.BlockSpec((1, 128), lambda i: (0, 0)),
    ],
    out_shape=[
        jax.ShapeDtypeStruct((N, FH), jnp.float32),
        jax.ShapeDtypeStruct((N, FH), jnp.float32),
        jax.ShapeDtypeStruct((N // RB, 1, RB), jnp.float32),
        jax.ShapeDtypeStruct((N // RB, 1, RB), jnp.float32),
        jax.ShapeDtypeStruct((1, 128), jnp.float32),
    ],
    scratch_shapes=[pltpu.SMEM((2,), jnp.float32)],
)


def _lrelu(v):
    return jnp.where(v > 0, v, 0.2 * v)


def _sc_body(as5_ref, ad5_ref, m_ref, src_ref, dst_ref, h0_ref, h1_ref,
             att_ref, out0_ref, out1_ref,
             asv, adv, mv, sv_, dv_, ld, r0, r1, nb, s3v, d3v, nvb,
             sden, agg, sem):
    c = lax.axis_index("c")
    s = lax.axis_index("s")
    wid = c * NS + s
    iot = lax.iota(jnp.int32, 16)

    for i in range(N // RB):
        pltpu.sync_copy(as5_ref.at[i, 0], asv.at[pl.ds(i * RB, RB)])
        pltpu.sync_copy(ad5_ref.at[i, 0], adv.at[pl.ds(i * RB, RB)])
    pltpu.sync_copy(m_ref.at[0], mv)
    pltpu.sync_copy(src_ref.at[pl.ds(s * EPT, EPT)], sv_)
    pltpu.sync_copy(dst_ref.at[pl.ds(s * EPT, EPT)], dv_)

    @pl.loop(0, 80)
    def _(k):
        z = jnp.zeros((16,), jnp.float32)
        for j in range(8):
            ld[k, pl.ds(j * 16, 16)] = z
            r0[k, pl.ds(j * 16, 16)] = z

    @pl.when(s == 0)
    def _():
        pltpu.sync_copy(ld, sden)

    def zero_agg():
        @pl.when(s < 15)
        def _():
            @pl.loop(0, 4)
            def _(q):
                pltpu.sync_copy(r0, agg.at[pl.ds(s * 320 + q * 80, 80)])

        @pl.when(s == 15)
        def _():
            pltpu.sync_copy(r0, agg.at[pl.ds(4800, 80)])
            pltpu.sync_copy(r0, agg.at[pl.ds(4880, 80)])
            pltpu.sync_copy(r0.at[pl.ds(0, 48)], agg.at[pl.ds(4960, 48)])

    zero_agg()
    plsc.subcore_barrier()

    m_l = _lrelu(mv[pl.ds(0, 16)])

    def edge_group(svec, dvec):
        e = plsc.load_gather(asv, [svec]) + plsc.load_gather(adv, [dvec])
        return jnp.exp(_lrelu(e) - m_l)

    @pl.loop(0, EPT // 16)
    def _(k):
        sl = pl.ds(k * 16, 16)
        dvec = dv_[sl]
        ex = edge_group(sv_[sl], dvec)
        plsc.addupdate_scatter(
            ld, [lax.shift_right_logical(dvec, 7), jnp.bitwise_and(dvec, 127)],
            ex)

    for g in range(5):
        pltpu.sync_copy(ld.at[pl.ds(g * 16, 16)], sden.at[iot + g * 16],
                        add=True)
    plsc.subcore_barrier()
    pltpu.sync_copy(sden, ld)

    def denom_group(dvec):
        return plsc.load_gather(
            ld, [lax.shift_right_logical(dvec, 7), jnp.bitwise_and(dvec, 127)])

    def phase2(h_ref, p):
        @pl.loop(0, NCH)
        def _(k):
            for g in range(5):
                sl = pl.ds(k * C2 + g * 16, 16)
                dvec = dv_[sl]
                ex = edge_group(sv_[sl], dvec)
                nb[pl.ds(g * 16, 16)] = ex / denom_group(dvec)
            pltpu.async_copy(h_ref.at[sv_.at[pl.ds(k * C2, C2)]], r0,
                             sem).wait()

            @pl.loop(0, C2)
            def _(i):
                w = plsc.load_gather(nb, [jnp.full((16,), i, jnp.int32)])
                for j in range(8):
                    sl = pl.ds(j * 16, 16)
                    r0[i, sl] = r0[i, sl] * w

            for g in range(5):
                dvec = dv_[pl.ds(k * C2 + g * 16, 16)]
                dloc = dvec - p * NP
                ok = jnp.logical_and(dloc >= 0, dloc < NP)
                dloc = jnp.where(ok, dloc, NP)
                pltpu.sync_copy(r0.at[pl.ds(g * 16, 16)], agg.at[dloc],
                                add=True)

    def phase3():
        @pl.when(wid < 25)
        def _():
            pltpu.sync_copy(src_ref.at[pl.ds(wid * E3, E3)], s3v)
            pltpu.sync_copy(dst_ref.at[pl.ds(wid * E3, E3)], d3v)

            @pl.loop(0, E3 // 128)
            def _(j):
                descs = []
                for g in range(8):
                    sl = pl.ds(j * 128 + g * 16, 16)
                    svec = s3v[sl]
                    dvec = d3v[sl]
                    ex = edge_group(svec, dvec)
                    nvb[pl.ds(g * 16, 16)] = ex / denom_group(dvec)
                    fvec = svec * N + dvec
                    descs.append(
                        pltpu.async_copy(nvb.at[pl.ds(g * 16, 16)],
                                         att_ref.at[fvec], sem))
                for d in descs:
                    d.wait()

    def phase4(h_ref, out_ref, p):
        def do_chunk(glob, n):
            loc = glob - p * NP
            pltpu.sync_copy(agg.at[pl.ds(loc, n)], r0.at[pl.ds(0, n)])
            pltpu.sync_copy(h_ref.at[pl.ds(glob, n)], r1.at[pl.ds(0, n)])

            @pl.loop(0, n)
            def _(i):
                nloc = jnp.full((16,), glob + i, jnp.int32)
                dn16 = plsc.load_gather(
                    ld, [lax.shift_right_logical(nloc, 7),
                         jnp.bitwise_and(nloc, 127)])
                keep = dn16 > 0.0
                for j in range(8):
                    sl = pl.ds(j * 16, 16)
                    r0[i, sl] = jnp.where(keep, r0[i, sl], r1[i, sl])

            pltpu.sync_copy(r0.at[pl.ds(0, n)], out_ref.at[pl.ds(glob, n)])

        base0 = p * NP

        @pl.when(s < 15)
        def _():
            @pl.loop(0, 4)
            def _(q):
                do_chunk(base0 + s * 320 + q * 80, 80)

        @pl.when(s == 15)
        def _():
            do_chunk(base0 + 4800, 80)
            do_chunk(base0 + 4880, 80)
            do_chunk(base0 + 4960, 40)

    for p in range(2):
        @pl.when(c == 0)
        def _():
            phase2(h0_ref, p)

        @pl.when(c == 1)
        def _():
            phase2(h1_ref, p)

        if p == 0:
            phase3()
        plsc.subcore_barrier()

        @pl.when(c == 0)
        def _():
            phase4(h0_ref, out0_ref, p)

        @pl.when(c == 1)
        def _():
            phase4(h1_ref, out1_ref, p)

        if p == 0:
            plsc.subcore_barrier()

            @pl.loop(0, 80)
            def _(k):
                z = jnp.zeros((16,), jnp.float32)
                for j in range(8):
                    r0[k, pl.ds(j * 16, 16)] = z

            zero_agg()
            plsc.subcore_barrier()


@functools.cache
def _sc_call():
    return pl.kernel(
        _sc_body,
        out_type=(
            jax.ShapeDtypeStruct((N, FH), jnp.float32),
            jax.ShapeDtypeStruct((N, FH), jnp.float32),
        ),
        mesh=plsc.VectorSubcoreMesh(core_axis_name="c", subcore_axis_name="s",
                                    num_cores=NC, num_subcores=NS),
        compiler_params=pltpu.CompilerParams(needs_layout_passes=False),
        scratch_types=[
            pltpu.VMEM((N,), jnp.float32),
            pltpu.VMEM((N,), jnp.float32),
            pltpu.VMEM((128,), jnp.float32),
            pltpu.VMEM((EPT,), jnp.int32),
            pltpu.VMEM((EPT,), jnp.int32),
            pltpu.VMEM((80, 128), jnp.float32),
            pltpu.VMEM((C2, FH), jnp.float32),
            pltpu.VMEM((C2, FH), jnp.float32),
            pltpu.VMEM((C2,), jnp.float32),
            pltpu.VMEM((E3,), jnp.int32),
            pltpu.VMEM((E3,), jnp.int32),
            pltpu.VMEM((128,), jnp.float32),
            pltpu.VMEM_SHARED((80, 128), jnp.float32),
            pltpu.VMEM_SHARED((NP + 8, FH), jnp.float32),
            pltpu.SemaphoreType.DMA,
        ],
    )


@jax.jit
def kernel(x, edge_index, W, att_src, att_dst):
    h0, h1, as5, ad5, mz = _tc_call(x, W, att_src, att_dst)
    src = edge_index[0]
    dst = edge_index[1]
    att_state = jax.new_ref(jnp.zeros((N * N,), jnp.float32))
    out0, out1 = _sc_call()(as5, ad5, mz, src, dst, h0, h1, att_state)
    att = att_state[...].reshape(N, N)
    out = jnp.concatenate([out0, out1], axis=1)
    return out, att


# speedup vs baseline: 3.5487x; 1.0823x over previous
import functools

import jax
import jax.numpy as jnp
from jax import lax
from jax.experimental import pallas as pl
from jax.experimental.pallas import tpu as pltpu
from jax.experimental.pallas import tpu_sc as plsc

N = 10000
NP = N // 2
E = 160000
F = 256
FH = 128
NC = 2
NS = 16
RB = 2000
EPT = E // NS
C2 = 80
NCH = EPT // C2
E3 = E // 25


def _tc_body(x_ref, w_ref, as_ref, ad_ref, h0_ref, h1_ref,
             as5_ref, ad5_ref, m_ref, acc_ref):
    i = pl.program_id(0)
    h = lax.dot_general(x_ref[...], w_ref[...], (((1,), (1,)), ((), ())),
                        preferred_element_type=jnp.float32)
    h0_ref[...] = h[:, :FH]
    h1_ref[...] = h[:, FH:]
    a_s = lax.dot_general(as_ref[...], h, (((1,), (1,)), ((), ())),
                          preferred_element_type=jnp.float32)
    a_d = lax.dot_general(ad_ref[...], h, (((1,), (1,)), ((), ())),
                          preferred_element_type=jnp.float32)
    as5_ref[0, 0, :] = a_s[0]
    ad5_ref[0, 0, :] = a_d[0]

    @pl.when(i == 0)
    def _():
        acc_ref[0] = -jnp.inf
        acc_ref[1] = -jnp.inf

    acc_ref[0] = jnp.maximum(acc_ref[0], jnp.max(a_s))
    acc_ref[1] = jnp.maximum(acc_ref[1], jnp.max(a_d))

    @pl.when(i == pl.num_programs(0) - 1)
    def _():
        m_ref[...] = jnp.full((1, 128), acc_ref[0] + acc_ref[1], jnp.float32)


_tc_call = pl.pallas_call(
    _tc_body,
    grid=(N // RB,),
    in_specs=[
        pl.BlockSpec((RB, F), lambda i: (i, 0)),
        pl.BlockSpec((F, F), lambda i: (0, 0)),
        pl.BlockSpec((1, F), lambda i: (0, 0)),
        pl.BlockSpec((1, F), lambda i: (0, 0)),
    ],
    out_specs=[
        pl.BlockSpec((RB, FH), lambda i: (i, 0)),
        pl.BlockSpec((RB, FH), lambda i: (i, 0)),
        pl.BlockSpec((1, 1, RB), lambda i: (i, 0, 0)),
        pl.BlockSpec((1, 1, RB), lambda i: (i, 0, 0)),
        pl.BlockSpec((1, 128), lambda i: (0, 0)),
    ],
    out_shape=[
        jax.ShapeDtypeStruct((N, FH), jnp.float32),
        jax.ShapeDtypeStruct((N, FH), jnp.float32),
        jax.ShapeDtypeStruct((N // RB, 1, RB), jnp.float32),
        jax.ShapeDtypeStruct((N // RB, 1, RB), jnp.float32),
        jax.ShapeDtypeStruct((1, 128), jnp.float32),
    ],
    scratch_shapes=[pltpu.SMEM((2,), jnp.float32)],
)


def _lrelu(v):
    return jnp.where(v > 0, v, 0.2 * v)


def _sc_body(as5_ref, ad5_ref, m_ref, src_ref, dst_ref, h0_ref, h1_ref,
             att_ref, out0_ref, out1_ref,
             asv, adv, mv, sv_, dv_, ld, r0, r1, nb, s3v, d3v, nvb,
             sden, agg, sem, sem_a, sem_b):
    c = lax.axis_index("c")
    s = lax.axis_index("s")
    wid = c * NS + s
    iot = lax.iota(jnp.int32, 16)

    for i in range(N // RB):
        pltpu.sync_copy(as5_ref.at[i, 0], asv.at[pl.ds(i * RB, RB)])
        pltpu.sync_copy(ad5_ref.at[i, 0], adv.at[pl.ds(i * RB, RB)])
    pltpu.sync_copy(m_ref.at[0], mv)
    pltpu.sync_copy(src_ref.at[pl.ds(s * EPT, EPT)], sv_)
    pltpu.sync_copy(dst_ref.at[pl.ds(s * EPT, EPT)], dv_)

    @pl.loop(0, 80)
    def _(k):
        z = jnp.zeros((16,), jnp.float32)
        for j in range(8):
            ld[k, pl.ds(j * 16, 16)] = z
            r0[k, pl.ds(j * 16, 16)] = z

    @pl.when(s == 0)
    def _():
        pltpu.sync_copy(ld, sden)

    def zero_agg():
        @pl.when(s < 15)
        def _():
            @pl.loop(0, 4)
            def _(q):
                pltpu.sync_copy(r0, agg.at[pl.ds(s * 320 + q * 80, 80)])

        @pl.when(s == 15)
        def _():
            pltpu.sync_copy(r0, agg.at[pl.ds(4800, 80)])
            pltpu.sync_copy(r0, agg.at[pl.ds(4880, 80)])
            pltpu.sync_copy(r0.at[pl.ds(0, 48)], agg.at[pl.ds(4960, 48)])

    zero_agg()
    plsc.subcore_barrier()

    m_l = _lrelu(mv[pl.ds(0, 16)])

    def edge_group(svec, dvec):
        e = plsc.load_gather(asv, [svec]) + plsc.load_gather(adv, [dvec])
        return jnp.exp(_lrelu(e) - m_l)

    @pl.loop(0, EPT // 16)
    def _(k):
        sl = pl.ds(k * 16, 16)
        dvec = dv_[sl]
        ex = edge_group(sv_[sl], dvec)
        plsc.addupdate_scatter(
            ld, [lax.shift_right_logical(dvec, 7), jnp.bitwise_and(dvec, 127)],
            ex)

    for g in range(5):
        pltpu.sync_copy(ld.at[pl.ds(g * 16, 16)], sden.at[iot + g * 16],
                        add=True)
    plsc.subcore_barrier()
    pltpu.sync_copy(sden, ld)

    def denom_group(dvec):
        return plsc.load_gather(
            ld, [lax.shift_right_logical(dvec, 7), jnp.bitwise_and(dvec, 127)])

    def phase2(h_ref, p):
        def gissue(k, buf, gsem):
            return pltpu.async_copy(h_ref.at[sv_.at[pl.ds(k * C2, C2)]], buf,
                                    gsem)

        def process(k, buf, desc):
            for g in range(5):
                sl = pl.ds(k * C2 + g * 16, 16)
                dvec = dv_[sl]
                ex = edge_group(sv_[sl], dvec)
                nb[pl.ds(g * 16, 16)] = ex / denom_group(dvec)
            desc.wait()

            @pl.loop(0, C2)
            def _(i):
                w = plsc.load_gather(nb, [jnp.full((16,), i, jnp.int32)])
                for j in range(8):
                    sl = pl.ds(j * 16, 16)
                    buf[i, sl] = buf[i, sl] * w

            for g in range(5):
                dvec = dv_[pl.ds(k * C2 + g * 16, 16)]
                dloc = dvec - p * NP
                ok = jnp.logical_and(dloc >= 0, dloc < NP)
                dloc = jnp.where(ok, dloc, NP)
                pltpu.sync_copy(buf.at[pl.ds(g * 16, 16)], agg.at[dloc],
                                add=True)

        @pl.loop(0, NCH // 2)
        def _(kk):
            k = kk * 2
            da = gissue(k, r0, sem_a)
            db = gissue(k + 1, r1, sem_b)
            process(k, r0, da)
            process(k + 1, r1, db)

        k_last = NCH - 1
        process(k_last, r0, gissue(k_last, r0, sem_a))

    def phase3():
        @pl.when(wid < 25)
        def _():
            pltpu.sync_copy(src_ref.at[pl.ds(wid * E3, E3)], s3v)
            pltpu.sync_copy(dst_ref.at[pl.ds(wid * E3, E3)], d3v)

            @pl.loop(0, E3 // 128)
            def _(j):
                descs = []
                for g in range(8):
                    sl = pl.ds(j * 128 + g * 16, 16)
                    svec = s3v[sl]
                    dvec = d3v[sl]
                    ex = edge_group(svec, dvec)
                    nvb[pl.ds(g * 16, 16)] = ex / denom_group(dvec)
                    fvec = svec * N + dvec
                    descs.append(
                        pltpu.async_copy(nvb.at[pl.ds(g * 16, 16)],
                                         att_ref.at[fvec], sem))
                for d in descs:
                    d.wait()

    def phase4(h_ref, out_ref, p):
        def do_chunk(glob, n):
            loc = glob - p * NP
            pltpu.sync_copy(agg.at[pl.ds(loc, n)], r0.at[pl.ds(0, n)])
            pltpu.sync_copy(h_ref.at[pl.ds(glob, n)], r1.at[pl.ds(0, n)])

            @pl.loop(0, n)
            def _(i):
                nloc = jnp.full((16,), glob + i, jnp.int32)
                dn16 = plsc.load_gather(
                    ld, [lax.shift_right_logical(nloc, 7),
                         jnp.bitwise_and(nloc, 127)])
                keep = dn16 > 0.0
                for j in range(8):
                    sl = pl.ds(j * 16, 16)
                    r0[i, sl] = jnp.where(keep, r0[i, sl], r1[i, sl])

            pltpu.sync_copy(r0.at[pl.ds(0, n)], out_ref.at[pl.ds(glob, n)])

        base0 = p * NP

        @pl.when(s < 15)
        def _():
            @pl.loop(0, 4)
            def _(q):
                do_chunk(base0 + s * 320 + q * 80, 80)

        @pl.when(s == 15)
        def _():
            do_chunk(base0 + 4800, 80)
            do_chunk(base0 + 4880, 80)
            do_chunk(base0 + 4960, 40)

    for p in range(2):
        @pl.when(c == 0)
        def _():
            phase2(h0_ref, p)

        @pl.when(c == 1)
        def _():
            phase2(h1_ref, p)

        if p == 0:
            phase3()
        plsc.subcore_barrier()

        @pl.when(c == 0)
        def _():
            phase4(h0_ref, out0_ref, p)

        @pl.when(c == 1)
        def _():
            phase4(h1_ref, out1_ref, p)

        if p == 0:
            plsc.subcore_barrier()

            @pl.loop(0, 80)
            def _(k):
                z = jnp.zeros((16,), jnp.float32)
                for j in range(8):
                    r0[k, pl.ds(j * 16, 16)] = z

            zero_agg()
            plsc.subcore_barrier()


@functools.cache
def _sc_call():
    return pl.kernel(
        _sc_body,
        out_type=(
            jax.ShapeDtypeStruct((N, FH), jnp.float32),
            jax.ShapeDtypeStruct((N, FH), jnp.float32),
        ),
        mesh=plsc.VectorSubcoreMesh(core_axis_name="c", subcore_axis_name="s",
                                    num_cores=NC, num_subcores=NS),
        compiler_params=pltpu.CompilerParams(needs_layout_passes=False),
        scratch_types=[
            pltpu.VMEM((N,), jnp.float32),
            pltpu.VMEM((N,), jnp.float32),
            pltpu.VMEM((128,), jnp.float32),
            pltpu.VMEM((EPT,), jnp.int32),
            pltpu.VMEM((EPT,), jnp.int32),
            pltpu.VMEM((80, 128), jnp.float32),
            pltpu.VMEM((C2, FH), jnp.float32),
            pltpu.VMEM((C2, FH), jnp.float32),
            pltpu.VMEM((C2,), jnp.float32),
            pltpu.VMEM((E3,), jnp.int32),
            pltpu.VMEM((E3,), jnp.int32),
            pltpu.VMEM((128,), jnp.float32),
            pltpu.VMEM_SHARED((80, 128), jnp.float32),
            pltpu.VMEM_SHARED((NP + 8, FH), jnp.float32),
            pltpu.SemaphoreType.DMA,
            pltpu.SemaphoreType.DMA,
            pltpu.SemaphoreType.DMA,
        ],
    )


@jax.jit
def kernel(x, edge_index, W, att_src, att_dst):
    h0, h1, as5, ad5, mz = _tc_call(x, W, att_src, att_dst)
    src = edge_index[0]
    dst = edge_index[1]
    att_state = jax.new_ref(jnp.zeros((N * N,), jnp.float32))
    out0, out1 = _sc_call()(as5, ad5, mz, src, dst, h0, h1, att_state)
    att = att_state[...].reshape(N, N)
    out = jnp.concatenate([out0, out1], axis=1)
    return out, att


# speedup vs baseline: 4.0526x; 1.1420x over previous
import functools

import jax
import jax.numpy as jnp
from jax import lax
from jax.experimental import pallas as pl
from jax.experimental.pallas import tpu as pltpu
from jax.experimental.pallas import tpu_sc as plsc

N = 10000
NP = N // 2
E = 160000
F = 256
FH = 128
NC = 2
NS = 16
RB = 2000
EPT = E // NS
C2 = 80
NCH = EPT // C2
E3 = E // 25


def _tc_body(x_ref, w_ref, as_ref, ad_ref, h0_ref, h1_ref,
             as5_ref, ad5_ref, m_ref, acc_ref):
    i = pl.program_id(0)
    h = lax.dot_general(x_ref[...], w_ref[...], (((1,), (1,)), ((), ())),
                        preferred_element_type=jnp.float32)
    h0_ref[...] = h[:, :FH]
    h1_ref[...] = h[:, FH:]
    a_s = lax.dot_general(as_ref[...], h, (((1,), (1,)), ((), ())),
                          preferred_element_type=jnp.float32)
    a_d = lax.dot_general(ad_ref[...], h, (((1,), (1,)), ((), ())),
                          preferred_element_type=jnp.float32)
    as5_ref[0, 0, :] = a_s[0]
    ad5_ref[0, 0, :] = a_d[0]

    @pl.when(i == 0)
    def _():
        acc_ref[0] = -jnp.inf
        acc_ref[1] = -jnp.inf

    acc_ref[0] = jnp.maximum(acc_ref[0], jnp.max(a_s))
    acc_ref[1] = jnp.maximum(acc_ref[1], jnp.max(a_d))

    @pl.when(i == pl.num_programs(0) - 1)
    def _():
        m_ref[...] = jnp.full((1, 128), acc_ref[0] + acc_ref[1], jnp.float32)


_tc_call = pl.pallas_call(
    _tc_body,
    grid=(N // RB,),
    in_specs=[
        pl.BlockSpec((RB, F), lambda i: (i, 0)),
        pl.BlockSpec((F, F), lambda i: (0, 0)),
        pl.BlockSpec((1, F), lambda i: (0, 0)),
        pl.BlockSpec((1, F), lambda i: (0, 0)),
    ],
    out_specs=[
        pl.BlockSpec((RB, FH), lambda i: (i, 0)),
        pl.BlockSpec((RB, FH), lambda i: (i, 0)),
        pl.BlockSpec((1, 1, RB), lambda i: (i, 0, 0)),
        pl.BlockSpec((1, 1, RB), lambda i: (i, 0, 0)),
        pl.BlockSpec((1, 128), lambda i: (0, 0)),
    ],
    out_shape=[
        jax.ShapeDtypeStruct((N, FH), jnp.float32),
        jax.ShapeDtypeStruct((N, FH), jnp.float32),
        jax.ShapeDtypeStruct((N // RB, 1, RB), jnp.float32),
        jax.ShapeDtypeStruct((N // RB, 1, RB), jnp.float32),
        jax.ShapeDtypeStruct((1, 128), jnp.float32),
    ],
    scratch_shapes=[pltpu.SMEM((2,), jnp.float32)],
)


def _lrelu(v):
    return jnp.where(v > 0, v, 0.2 * v)


def _sc_body(as5_ref, ad5_ref, m_ref, src_ref, dst_ref, h0_ref, h1_ref,
             att_ref, out0_ref, out1_ref,
             asv, adv, mv, sv_, dv_, ld, r0, r1, nb, s3v, d3v, nvb,
             sden, agg, sem, sem_a, sem_b):
    c = lax.axis_index("c")
    s = lax.axis_index("s")
    wid = c * NS + s
    iot = lax.iota(jnp.int32, 16)

    for i in range(N // RB):
        pltpu.sync_copy(as5_ref.at[i, 0], asv.at[pl.ds(i * RB, RB)])
        pltpu.sync_copy(ad5_ref.at[i, 0], adv.at[pl.ds(i * RB, RB)])
    pltpu.sync_copy(m_ref.at[0], mv)
    pltpu.sync_copy(src_ref.at[pl.ds(s * EPT, EPT)], sv_)
    pltpu.sync_copy(dst_ref.at[pl.ds(s * EPT, EPT)], dv_)

    @pl.loop(0, 80)
    def _(k):
        z = jnp.zeros((16,), jnp.float32)
        for j in range(8):
            ld[k, pl.ds(j * 16, 16)] = z
            r0[k, pl.ds(j * 16, 16)] = z

    @pl.when(s == 0)
    def _():
        pltpu.sync_copy(ld, sden)

    def zero_agg():
        @pl.when(s < 15)
        def _():
            @pl.loop(0, 4)
            def _(q):
                pltpu.sync_copy(r0, agg.at[pl.ds(s * 320 + q * 80, 80)])

        @pl.when(s == 15)
        def _():
            pltpu.sync_copy(r0, agg.at[pl.ds(4800, 80)])
            pltpu.sync_copy(r0, agg.at[pl.ds(4880, 80)])
            pltpu.sync_copy(r0.at[pl.ds(0, 48)], agg.at[pl.ds(4960, 48)])

    zero_agg()
    plsc.subcore_barrier()

    m_l = _lrelu(mv[pl.ds(0, 16)])

    def edge_group(svec, dvec):
        e = plsc.load_gather(asv, [svec]) + plsc.load_gather(adv, [dvec])
        return jnp.exp(_lrelu(e) - m_l)

    @pl.loop(0, EPT // 16)
    def _(k):
        sl = pl.ds(k * 16, 16)
        dvec = dv_[sl]
        ex = edge_group(sv_[sl], dvec)
        plsc.addupdate_scatter(
            ld, [lax.shift_right_logical(dvec, 7), jnp.bitwise_and(dvec, 127)],
            ex)

    for g in range(5):
        pltpu.sync_copy(ld.at[pl.ds(g * 16, 16)], sden.at[iot + g * 16],
                        add=True)
    plsc.subcore_barrier()
    pltpu.sync_copy(sden, ld)

    def denom_group(dvec):
        return plsc.load_gather(
            ld, [lax.shift_right_logical(dvec, 7), jnp.bitwise_and(dvec, 127)])

    def phase2(h_ref, p):
        def gissue(k, buf, gsem):
            return pltpu.async_copy(h_ref.at[sv_.at[pl.ds(k * C2, C2)]], buf,
                                    gsem)

        def process(k, buf, desc):
            for g in range(5):
                sl = pl.ds(k * C2 + g * 16, 16)
                dvec = dv_[sl]
                ex = edge_group(sv_[sl], dvec)
                nb[pl.ds(g * 16, 16)] = ex / denom_group(dvec)
            desc.wait()

            @plsc.parallel_loop(0, C2, unroll=4)
            def _(i):
                w = plsc.load_gather(nb, [jnp.full((16,), i, jnp.int32)])
                for j in range(8):
                    sl = pl.ds(j * 16, 16)
                    buf[i, sl] = buf[i, sl] * w

            sdescs = []
            for g in range(5):
                dvec = dv_[pl.ds(k * C2 + g * 16, 16)]
                dloc = dvec - p * NP
                ok = jnp.logical_and(dloc >= 0, dloc < NP)
                dloc = jnp.where(ok, dloc, NP)
                sdescs.append(
                    pltpu.async_copy(buf.at[pl.ds(g * 16, 16)], agg.at[dloc],
                                     sem, add=True))
            for d in sdescs:
                d.wait()

        @pl.loop(0, NCH // 2)
        def _(kk):
            k = kk * 2
            da = gissue(k, r0, sem_a)
            db = gissue(k + 1, r1, sem_b)
            process(k, r0, da)
            process(k + 1, r1, db)

        k_last = NCH - 1
        process(k_last, r0, gissue(k_last, r0, sem_a))

    def phase3():
        @pl.when(wid < 25)
        def _():
            pltpu.sync_copy(src_ref.at[pl.ds(wid * E3, E3)], s3v)
            pltpu.sync_copy(dst_ref.at[pl.ds(wid * E3, E3)], d3v)

            @pl.loop(0, E3 // 128)
            def _(j):
                descs = []
                for g in range(8):
                    sl = pl.ds(j * 128 + g * 16, 16)
                    svec = s3v[sl]
                    dvec = d3v[sl]
                    ex = edge_group(svec, dvec)
                    nvb[pl.ds(g * 16, 16)] = ex / denom_group(dvec)
                    fvec = svec * N + dvec
                    descs.append(
                        pltpu.async_copy(nvb.at[pl.ds(g * 16, 16)],
                                         att_ref.at[fvec], sem))
                for d in descs:
                    d.wait()

    def phase4(h_ref, out_ref, p):
        def do_chunk(glob, n):
            loc = glob - p * NP
            pltpu.sync_copy(agg.at[pl.ds(loc, n)], r0.at[pl.ds(0, n)])
            pltpu.sync_copy(h_ref.at[pl.ds(glob, n)], r1.at[pl.ds(0, n)])

            @pl.loop(0, n)
            def _(i):
                nloc = jnp.full((16,), glob + i, jnp.int32)
                dn16 = plsc.load_gather(
                    ld, [lax.shift_right_logical(nloc, 7),
                         jnp.bitwise_and(nloc, 127)])
                keep = dn16 > 0.0
                for j in range(8):
                    sl = pl.ds(j * 16, 16)
                    r0[i, sl] = jnp.where(keep, r0[i, sl], r1[i, sl])

            pltpu.sync_copy(r0.at[pl.ds(0, n)], out_ref.at[pl.ds(glob, n)])

        base0 = p * NP

        @pl.when(s < 15)
        def _():
            @pl.loop(0, 4)
            def _(q):
                do_chunk(base0 + s * 320 + q * 80, 80)

        @pl.when(s == 15)
        def _():
            do_chunk(base0 + 4800, 80)
            do_chunk(base0 + 4880, 80)
            do_chunk(base0 + 4960, 40)

    for p in range(2):
        @pl.when(c == 0)
        def _():
            phase2(h0_ref, p)

        @pl.when(c == 1)
        def _():
            phase2(h1_ref, p)

        if p == 0:
            phase3()
        plsc.subcore_barrier()

        @pl.when(c == 0)
        def _():
            phase4(h0_ref, out0_ref, p)

        @pl.when(c == 1)
        def _():
            phase4(h1_ref, out1_ref, p)

        if p == 0:
            plsc.subcore_barrier()

            @pl.loop(0, 80)
            def _(k):
                z = jnp.zeros((16,), jnp.float32)
                for j in range(8):
                    r0[k, pl.ds(j * 16, 16)] = z

            zero_agg()
            plsc.subcore_barrier()


@functools.cache
def _sc_call():
    return pl.kernel(
        _sc_body,
        out_type=(
            jax.ShapeDtypeStruct((N, FH), jnp.float32),
            jax.ShapeDtypeStruct((N, FH), jnp.float32),
        ),
        mesh=plsc.VectorSubcoreMesh(core_axis_name="c", subcore_axis_name="s",
                                    num_cores=NC, num_subcores=NS),
        compiler_params=pltpu.CompilerParams(needs_layout_passes=False),
        scratch_types=[
            pltpu.VMEM((N,), jnp.float32),
            pltpu.VMEM((N,), jnp.float32),
            pltpu.VMEM((128,), jnp.float32),
            pltpu.VMEM((EPT,), jnp.int32),
            pltpu.VMEM((EPT,), jnp.int32),
            pltpu.VMEM((80, 128), jnp.float32),
            pltpu.VMEM((C2, FH), jnp.float32),
            pltpu.VMEM((C2, FH), jnp.float32),
            pltpu.VMEM((C2,), jnp.float32),
            pltpu.VMEM((E3,), jnp.int32),
            pltpu.VMEM((E3,), jnp.int32),
            pltpu.VMEM((128,), jnp.float32),
            pltpu.VMEM_SHARED((80, 128), jnp.float32),
            pltpu.VMEM_SHARED((NP + 8, FH), jnp.float32),
            pltpu.SemaphoreType.DMA,
            pltpu.SemaphoreType.DMA,
            pltpu.SemaphoreType.DMA,
        ],
    )


@jax.jit
def kernel(x, edge_index, W, att_src, att_dst):
    h0, h1, as5, ad5, mz = _tc_call(x, W, att_src, att_dst)
    src = edge_index[0]
    dst = edge_index[1]
    att_state = jax.new_ref(jnp.zeros((N * N,), jnp.float32))
    out0, out1 = _sc_call()(as5, ad5, mz, src, dst, h0, h1, att_state)
    att = att_state[...].reshape(N, N)
    out = jnp.concatenate([out0, out1], axis=1)
    return out, att


# speedup vs baseline: 4.2139x; 1.0398x over previous
import functools

import jax
import jax.numpy as jnp
from jax import lax
from jax.experimental import pallas as pl
from jax.experimental.pallas import tpu as pltpu
from jax.experimental.pallas import tpu_sc as plsc

N = 10000
NP = N // 2
E = 160000
F = 256
FH = 128
NC = 2
NS = 16
RB = 2000
EPT = E // NS
C2 = 80
NCH = EPT // C2
E3 = E // 25


def _tc_body(x_ref, w_ref, as_ref, ad_ref, h0_ref, h1_ref,
             as5_ref, ad5_ref, m_ref, acc_ref):
    i = pl.program_id(0)
    h = lax.dot_general(x_ref[...], w_ref[...], (((1,), (1,)), ((), ())),
                        preferred_element_type=jnp.float32)
    h0_ref[...] = h[:, :FH]
    h1_ref[...] = h[:, FH:]
    a_s = lax.dot_general(as_ref[...], h, (((1,), (1,)), ((), ())),
                          preferred_element_type=jnp.float32)
    a_d = lax.dot_general(ad_ref[...], h, (((1,), (1,)), ((), ())),
                          preferred_element_type=jnp.float32)
    as5_ref[0, 0, :] = a_s[0]
    ad5_ref[0, 0, :] = a_d[0]

    @pl.when(i == 0)
    def _():
        acc_ref[0] = -jnp.inf
        acc_ref[1] = -jnp.inf

    acc_ref[0] = jnp.maximum(acc_ref[0], jnp.max(a_s))
    acc_ref[1] = jnp.maximum(acc_ref[1], jnp.max(a_d))

    @pl.when(i == pl.num_programs(0) - 1)
    def _():
        m_ref[...] = jnp.full((1, 128), acc_ref[0] + acc_ref[1], jnp.float32)


_tc_call = pl.pallas_call(
    _tc_body,
    grid=(N // RB,),
    in_specs=[
        pl.BlockSpec((RB, F), lambda i: (i, 0)),
        pl.BlockSpec((F, F), lambda i: (0, 0)),
        pl.BlockSpec((1, F), lambda i: (0, 0)),
        pl.BlockSpec((1, F), lambda i: (0, 0)),
    ],
    out_specs=[
        pl.BlockSpec((RB, FH), lambda i: (i, 0)),
        pl.BlockSpec((RB, FH), lambda i: (i, 0)),
        pl.BlockSpec((1, 1, RB), lambda i: (i, 0, 0)),
        pl.BlockSpec((1, 1, RB), lambda i: (i, 0, 0)),
        pl.BlockSpec((1, 128), lambda i: (0, 0)),
    ],
    out_shape=[
        jax.ShapeDtypeStruct((N, FH), jnp.float32),
        jax.ShapeDtypeStruct((N, FH), jnp.float32),
        jax.ShapeDtypeStruct((N // RB, 1, RB), jnp.float32),
        jax.ShapeDtypeStruct((N // RB, 1, RB), jnp.float32),
        jax.ShapeDtypeStruct((1, 128), jnp.float32),
    ],
    scratch_shapes=[pltpu.SMEM((2,), jnp.float32)],
)


def _lrelu(v):
    return jnp.where(v > 0, v, 0.2 * v)


def _sc_body(as5_ref, ad5_ref, m_ref, src_ref, dst_ref, h0_ref, h1_ref,
             att_ref, out0_ref, out1_ref,
             asv, adv, mv, sv_, dv_, ld, r0, r1, nb, s3v, d3v,
             dl0, dl1, fb0, fb1, nv0, nv1,
             sden, agg, sem, sem_a, sem_b):
    c = lax.axis_index("c")
    s = lax.axis_index("s")
    wid = c * NS + s
    iot = lax.iota(jnp.int32, 16)

    for i in range(N // RB):
        pltpu.sync_copy(as5_ref.at[i, 0], asv.at[pl.ds(i * RB, RB)])
        pltpu.sync_copy(ad5_ref.at[i, 0], adv.at[pl.ds(i * RB, RB)])
    pltpu.sync_copy(m_ref.at[0], mv)
    pltpu.sync_copy(src_ref.at[pl.ds(s * EPT, EPT)], sv_)
    pltpu.sync_copy(dst_ref.at[pl.ds(s * EPT, EPT)], dv_)

    @pl.loop(0, 80)
    def _(k):
        z = jnp.zeros((16,), jnp.float32)
        for j in range(8):
            ld[k, pl.ds(j * 16, 16)] = z
            r0[k, pl.ds(j * 16, 16)] = z

    @pl.when(s == 0)
    def _():
        pltpu.sync_copy(ld, sden)

    def zero_agg():
        @pl.when(s < 15)
        def _():
            @pl.loop(0, 4)
            def _(q):
                pltpu.sync_copy(r0, agg.at[pl.ds(s * 320 + q * 80, 80)])

        @pl.when(s == 15)
        def _():
            pltpu.sync_copy(r0, agg.at[pl.ds(4800, 80)])
            pltpu.sync_copy(r0, agg.at[pl.ds(4880, 80)])
            pltpu.sync_copy(r0.at[pl.ds(0, 48)], agg.at[pl.ds(4960, 48)])

    zero_agg()
    plsc.subcore_barrier()

    m_l = _lrelu(mv[pl.ds(0, 16)])

    def edge_group(svec, dvec):
        e = plsc.load_gather(asv, [svec]) + plsc.load_gather(adv, [dvec])
        return jnp.exp(_lrelu(e) - m_l)

    @plsc.parallel_loop(0, EPT // 16, unroll=2)
    def _(k):
        sl = pl.ds(k * 16, 16)
        dvec = dv_[sl]
        ex = edge_group(sv_[sl], dvec)
        plsc.addupdate_scatter(
            ld, [lax.shift_right_logical(dvec, 7), jnp.bitwise_and(dvec, 127)],
            ex)

    for g in range(5):
        pltpu.sync_copy(ld.at[pl.ds(g * 16, 16)], sden.at[iot + g * 16],
                        add=True)
    plsc.subcore_barrier()
    pltpu.sync_copy(sden, ld)

    def denom_group(dvec):
        return plsc.load_gather(
            ld, [lax.shift_right_logical(dvec, 7), jnp.bitwise_and(dvec, 127)])

    def phase2(h_ref, p):
        def gissue(k, buf, gsem):
            return pltpu.async_copy(h_ref.at[sv_.at[pl.ds(k * C2, C2)]], buf,
                                    gsem)

        def process(k, buf, desc, dlb):
            for g in range(5):
                sl = pl.ds(k * C2 + g * 16, 16)
                dvec = dv_[sl]
                ex = edge_group(sv_[sl], dvec)
                nb[pl.ds(g * 16, 16)] = ex / denom_group(dvec)
                dloc = dvec - p * NP
                ok = jnp.logical_and(dloc >= 0, dloc < NP)
                dlb[pl.ds(g * 16, 16)] = jnp.where(ok, dloc, NP)
            desc.wait()

            @plsc.parallel_loop(0, C2, unroll=4)
            def _(i):
                w = plsc.load_gather(nb, [jnp.full((16,), i, jnp.int32)])
                for j in range(8):
                    sl = pl.ds(j * 16, 16)
                    buf[i, sl] = buf[i, sl] * w

            return pltpu.async_copy(buf, agg.at[dlb], sem, add=True)

        @pl.loop(0, NCH // 2)
        def _(kk):
            k = kk * 2
            da = gissue(k, r0, sem_a)
            db = gissue(k + 1, r1, sem_b)
            wa = process(k, r0, da, dl0)
            wb = process(k + 1, r1, db, dl1)
            wa.wait()
            wb.wait()

        k_last = NCH - 1
        process(k_last, r0, gissue(k_last, r0, sem_a), dl0).wait()

    def phase3():
        @pl.when(wid < 25)
        def _():
            pltpu.sync_copy(src_ref.at[pl.ds(wid * E3, E3)], s3v)
            pltpu.sync_copy(dst_ref.at[pl.ds(wid * E3, E3)], d3v)

            @pl.loop(0, E3 // 256)
            def _(jj):
                descs = []
                for half, (fb, nv) in enumerate(((fb0, nv0), (fb1, nv1))):
                    j = jj * 2 + half
                    for g in range(8):
                        sl = pl.ds(j * 128 + g * 16, 16)
                        svec = s3v[sl]
                        dvec = d3v[sl]
                        ex = edge_group(svec, dvec)
                        nv[pl.ds(g * 16, 16)] = ex / denom_group(dvec)
                        fb[pl.ds(g * 16, 16)] = svec * N + dvec
                    descs.append(pltpu.async_copy(nv, att_ref.at[fb], sem))
                for d in descs:
                    d.wait()

    def phase4(h_ref, out_ref, p):
        def do_chunk(glob, n):
            loc = glob - p * NP
            pltpu.sync_copy(agg.at[pl.ds(loc, n)], r0.at[pl.ds(0, n)])
            pltpu.sync_copy(h_ref.at[pl.ds(glob, n)], r1.at[pl.ds(0, n)])

            @plsc.parallel_loop(0, n, unroll=4)
            def _(i):
                nloc = jnp.full((16,), glob + i, jnp.int32)
                dn16 = plsc.load_gather(
                    ld, [lax.shift_right_logical(nloc, 7),
                         jnp.bitwise_and(nloc, 127)])
                keep = dn16 > 0.0
                for j in range(8):
                    sl = pl.ds(j * 16, 16)
                    r0[i, sl] = jnp.where(keep, r0[i, sl], r1[i, sl])

            pltpu.sync_copy(r0.at[pl.ds(0, n)], out_ref.at[pl.ds(glob, n)])

        base0 = p * NP

        @pl.when(s < 15)
        def _():
            @pl.loop(0, 4)
            def _(q):
                do_chunk(base0 + s * 320 + q * 80, 80)

        @pl.when(s == 15)
        def _():
            do_chunk(base0 + 4800, 80)
            do_chunk(base0 + 4880, 80)
            do_chunk(base0 + 4960, 40)

    for p in range(2):
        @pl.when(c == 0)
        def _():
            phase2(h0_ref, p)

        @pl.when(c == 1)
        def _():
            phase2(h1_ref, p)

        if p == 0:
            phase3()
        plsc.subcore_barrier()

        @pl.when(c == 0)
        def _():
            phase4(h0_ref, out0_ref, p)

        @pl.when(c == 1)
        def _():
            phase4(h1_ref, out1_ref, p)

        if p == 0:
            plsc.subcore_barrier()

            @pl.loop(0, 80)
            def _(k):
                z = jnp.zeros((16,), jnp.float32)
                for j in range(8):
                    r0[k, pl.ds(j * 16, 16)] = z

            zero_agg()
            plsc.subcore_barrier()


@functools.cache
def _sc_call():
    return pl.kernel(
        _sc_body,
        out_type=(
            jax.ShapeDtypeStruct((N, FH), jnp.float32),
            jax.ShapeDtypeStruct((N, FH), jnp.float32),
        ),
        mesh=plsc.VectorSubcoreMesh(core_axis_name="c", subcore_axis_name="s",
                                    num_cores=NC, num_subcores=NS),
        compiler_params=pltpu.CompilerParams(needs_layout_passes=False),
        scratch_types=[
            pltpu.VMEM((N,), jnp.float32),
            pltpu.VMEM((N,), jnp.float32),
            pltpu.VMEM((128,), jnp.float32),
            pltpu.VMEM((EPT,), jnp.int32),
            pltpu.VMEM((EPT,), jnp.int32),
            pltpu.VMEM((80, 128), jnp.float32),
            pltpu.VMEM((C2, FH), jnp.float32),
            pltpu.VMEM((C2, FH), jnp.float32),
            pltpu.VMEM((C2,), jnp.float32),
            pltpu.VMEM((E3,), jnp.int32),
            pltpu.VMEM((E3,), jnp.int32),
            pltpu.VMEM((C2,), jnp.int32),
            pltpu.VMEM((C2,), jnp.int32),
            pltpu.VMEM((128,), jnp.int32),
            pltpu.VMEM((128,), jnp.int32),
            pltpu.VMEM((128,), jnp.float32),
            pltpu.VMEM((128,), jnp.float32),
            pltpu.VMEM_SHARED((80, 128), jnp.float32),
            pltpu.VMEM_SHARED((NP + 8, FH), jnp.float32),
            pltpu.SemaphoreType.DMA,
            pltpu.SemaphoreType.DMA,
            pltpu.SemaphoreType.DMA,
        ],
    )


@jax.jit
def kernel(x, edge_index, W, att_src, att_dst):
    h0, h1, as5, ad5, mz = _tc_call(x, W, att_src, att_dst)
    src = edge_index[0]
    dst = edge_index[1]
    att_state = jax.new_ref(jnp.zeros((N * N,), jnp.float32))
    out0, out1 = _sc_call()(as5, ad5, mz, src, dst, h0, h1, att_state)
    att = att_state[...].reshape(N, N)
    out = jnp.concatenate([out0, out1], axis=1)
    return out, att


# speedup vs baseline: 5.4203x; 1.2863x over previous
import functools

import jax
import jax.numpy as jnp
from jax import lax
from jax.experimental import pallas as pl
from jax.experimental.pallas import tpu as pltpu
from jax.experimental.pallas import tpu_sc as plsc

N = 10000
NP = N // 2
E = 160000
F = 256
FH = 128
NC = 2
NS = 16
RB = 2000
EPT = E // NS
C2 = 80
NCH = EPT // C2
E3 = E // 25


def _tc_body(x_ref, w_ref, as_ref, ad_ref, h0_ref, h1_ref,
             as5_ref, ad5_ref, m_ref, acc_ref):
    i = pl.program_id(0)
    h = lax.dot_general(x_ref[...], w_ref[...], (((1,), (1,)), ((), ())),
                        preferred_element_type=jnp.float32)
    h0_ref[...] = h[:, :FH]
    h1_ref[...] = h[:, FH:]
    a_s = lax.dot_general(as_ref[...], h, (((1,), (1,)), ((), ())),
                          preferred_element_type=jnp.float32)
    a_d = lax.dot_general(ad_ref[...], h, (((1,), (1,)), ((), ())),
                          preferred_element_type=jnp.float32)
    as5_ref[0, 0, :] = a_s[0]
    ad5_ref[0, 0, :] = a_d[0]

    @pl.when(i == 0)
    def _():
        acc_ref[0] = -jnp.inf
        acc_ref[1] = -jnp.inf

    acc_ref[0] = jnp.maximum(acc_ref[0], jnp.max(a_s))
    acc_ref[1] = jnp.maximum(acc_ref[1], jnp.max(a_d))

    @pl.when(i == pl.num_programs(0) - 1)
    def _():
        m_ref[...] = jnp.full((1, 128), acc_ref[0] + acc_ref[1], jnp.float32)


_tc_call = pl.pallas_call(
    _tc_body,
    grid=(N // RB,),
    in_specs=[
        pl.BlockSpec((RB, F), lambda i: (i, 0)),
        pl.BlockSpec((F, F), lambda i: (0, 0)),
        pl.BlockSpec((1, F), lambda i: (0, 0)),
        pl.BlockSpec((1, F), lambda i: (0, 0)),
    ],
    out_specs=[
        pl.BlockSpec((RB, FH), lambda i: (i, 0)),
        pl.BlockSpec((RB, FH), lambda i: (i, 0)),
        pl.BlockSpec((1, 1, RB), lambda i: (i, 0, 0)),
        pl.BlockSpec((1, 1, RB), lambda i: (i, 0, 0)),
        pl.BlockSpec((1, 128), lambda i: (0, 0)),
    ],
    out_shape=[
        jax.ShapeDtypeStruct((N, FH), jnp.float32),
        jax.ShapeDtypeStruct((N, FH), jnp.float32),
        jax.ShapeDtypeStruct((N // RB, 1, RB), jnp.float32),
        jax.ShapeDtypeStruct((N // RB, 1, RB), jnp.float32),
        jax.ShapeDtypeStruct((1, 128), jnp.float32),
    ],
    scratch_shapes=[pltpu.SMEM((2,), jnp.float32)],
)


def _lrelu(v):
    return jnp.where(v > 0, v, 0.2 * v)


def _sc_att_body(as5_ref, ad5_ref, m_ref, src_ref, dst_ref,
                 att_ref, den_ref,
                 asv, adv, mv, sv_, dv_, ld, s3v, d3v,
                 fb0, fb1, nv0, nv1, sden, sem):
    c = lax.axis_index("c")
    s = lax.axis_index("s")
    wid = c * NS + s
    iot = lax.iota(jnp.int32, 16)

    for i in range(N // RB):
        pltpu.sync_copy(as5_ref.at[i, 0], asv.at[pl.ds(i * RB, RB)])
        pltpu.sync_copy(ad5_ref.at[i, 0], adv.at[pl.ds(i * RB, RB)])
    pltpu.sync_copy(m_ref.at[0], mv)
    pltpu.sync_copy(src_ref.at[pl.ds(s * EPT, EPT)], sv_)
    pltpu.sync_copy(dst_ref.at[pl.ds(s * EPT, EPT)], dv_)

    @pl.loop(0, 80)
    def _(k):
        z = jnp.zeros((16,), jnp.float32)
        for j in range(8):
            ld[k, pl.ds(j * 16, 16)] = z

    @pl.when(s == 0)
    def _():
        pltpu.sync_copy(ld, sden)

    plsc.subcore_barrier()

    m_l = _lrelu(mv[pl.ds(0, 16)])

    def edge_group(svec, dvec):
        e = plsc.load_gather(asv, [svec]) + plsc.load_gather(adv, [dvec])
        return jnp.exp(_lrelu(e) - m_l)

    @plsc.parallel_loop(0, EPT // 16, unroll=2)
    def _(k):
        sl = pl.ds(k * 16, 16)
        dvec = dv_[sl]
        ex = edge_group(sv_[sl], dvec)
        plsc.addupdate_scatter(
            ld, [lax.shift_right_logical(dvec, 7), jnp.bitwise_and(dvec, 127)],
            ex)

    for g in range(5):
        pltpu.sync_copy(ld.at[pl.ds(g * 16, 16)], sden.at[iot + g * 16],
                        add=True)
    plsc.subcore_barrier()
    pltpu.sync_copy(sden, ld)

    @pl.when(jnp.logical_and(c == 0, s == 0))
    def _():
        pltpu.sync_copy(ld, den_ref)

    def denom_group(dvec):
        return plsc.load_gather(
            ld, [lax.shift_right_logical(dvec, 7), jnp.bitwise_and(dvec, 127)])

    @pl.when(wid < 25)
    def _():
        pltpu.sync_copy(src_ref.at[pl.ds(wid * E3, E3)], s3v)
        pltpu.sync_copy(dst_ref.at[pl.ds(wid * E3, E3)], d3v)

        @pl.loop(0, E3 // 256)
        def _(jj):
            descs = []
            for half, (fb, nv) in enumerate(((fb0, nv0), (fb1, nv1))):
                j = jj * 2 + half
                for g in range(8):
                    sl = pl.ds(j * 128 + g * 16, 16)
                    svec = s3v[sl]
                    dvec = d3v[sl]
                    ex = edge_group(svec, dvec)
                    nv[pl.ds(g * 16, 16)] = ex / denom_group(dvec)
                    fb[pl.ds(g * 16, 16)] = svec * N + dvec
                descs.append(pltpu.async_copy(nv, att_ref.at[fb], sem))
            for d in descs:
                d.wait()


@functools.cache
def _sc_att_call():
    return pl.kernel(
        _sc_att_body,
        out_type=jax.ShapeDtypeStruct((80, 128), jnp.float32),
        mesh=plsc.VectorSubcoreMesh(core_axis_name="c", subcore_axis_name="s",
                                    num_cores=NC, num_subcores=NS),
        compiler_params=pltpu.CompilerParams(needs_layout_passes=False),
        scratch_types=[
            pltpu.VMEM((N,), jnp.float32),
            pltpu.VMEM((N,), jnp.float32),
            pltpu.VMEM((128,), jnp.float32),
            pltpu.VMEM((EPT,), jnp.int32),
            pltpu.VMEM((EPT,), jnp.int32),
            pltpu.VMEM((80, 128), jnp.float32),
            pltpu.VMEM((E3,), jnp.int32),
            pltpu.VMEM((E3,), jnp.int32),
            pltpu.VMEM((128,), jnp.int32),
            pltpu.VMEM((128,), jnp.int32),
            pltpu.VMEM((128,), jnp.float32),
            pltpu.VMEM((128,), jnp.float32),
            pltpu.VMEM_SHARED((80, 128), jnp.float32),
            pltpu.SemaphoreType.DMA,
        ],
    )


def _sc_main_body(as5_ref, ad5_ref, m_ref, src_ref, dst_ref, den_ref,
                  h0_ref, h1_ref, out0_ref, out1_ref,
                  asv, adv, mv, sv_, dv_, ld, r0, r1, nb,
                  dl0, dl1, agg, sem, sem_a, sem_b):
    c = lax.axis_index("c")
    s = lax.axis_index("s")

    for i in range(N // RB):
        pltpu.sync_copy(as5_ref.at[i, 0], asv.at[pl.ds(i * RB, RB)])
        pltpu.sync_copy(ad5_ref.at[i, 0], adv.at[pl.ds(i * RB, RB)])
    pltpu.sync_copy(m_ref.at[0], mv)
    pltpu.sync_copy(src_ref.at[pl.ds(s * EPT, EPT)], sv_)
    pltpu.sync_copy(dst_ref.at[pl.ds(s * EPT, EPT)], dv_)
    pltpu.sync_copy(den_ref, ld)

    @pl.loop(0, 80)
    def _(k):
        z = jnp.zeros((16,), jnp.float32)
        for j in range(8):
            r0[k, pl.ds(j * 16, 16)] = z

    def zero_agg():
        @pl.when(s < 15)
        def _():
            @pl.loop(0, 4)
            def _(q):
                pltpu.sync_copy(r0, agg.at[pl.ds(s * 320 + q * 80, 80)])

        @pl.when(s == 15)
        def _():
            pltpu.sync_copy(r0, agg.at[pl.ds(4800, 80)])
            pltpu.sync_copy(r0, agg.at[pl.ds(4880, 80)])
            pltpu.sync_copy(r0.at[pl.ds(0, 48)], agg.at[pl.ds(4960, 48)])

    zero_agg()
    plsc.subcore_barrier()

    m_l = _lrelu(mv[pl.ds(0, 16)])

    def edge_group(svec, dvec):
        e = plsc.load_gather(asv, [svec]) + plsc.load_gather(adv, [dvec])
        return jnp.exp(_lrelu(e) - m_l)

    def denom_group(dvec):
        return plsc.load_gather(
            ld, [lax.shift_right_logical(dvec, 7), jnp.bitwise_and(dvec, 127)])

    def phase2(h_ref, p):
        def gissue(k, buf, gsem):
            return pltpu.async_copy(h_ref.at[sv_.at[pl.ds(k * C2, C2)]], buf,
                                    gsem)

        def process(k, buf, desc, dlb):
            for g in range(5):
                sl = pl.ds(k * C2 + g * 16, 16)
                dvec = dv_[sl]
                ex = edge_group(sv_[sl], dvec)
                nb[pl.ds(g * 16, 16)] = ex / denom_group(dvec)
                dloc = dvec - p * NP
                ok = jnp.logical_and(dloc >= 0, dloc < NP)
                dlb[pl.ds(g * 16, 16)] = jnp.where(ok, dloc, NP)
            desc.wait()

            @plsc.parallel_loop(0, C2, unroll=4)
            def _(i):
                w = plsc.load_gather(nb, [jnp.full((16,), i, jnp.int32)])
                for j in range(8):
                    sl = pl.ds(j * 16, 16)
                    buf[i, sl] = buf[i, sl] * w

            return pltpu.async_copy(buf, agg.at[dlb], sem, add=True)

        @pl.loop(0, NCH // 2)
        def _(kk):
            k = kk * 2
            da = gissue(k, r0, sem_a)
            db = gissue(k + 1, r1, sem_b)
            wa = process(k, r0, da, dl0)
            wb = process(k + 1, r1, db, dl1)
            wa.wait()
            wb.wait()

        k_last = NCH - 1
        process(k_last, r0, gissue(k_last, r0, sem_a), dl0).wait()

    def phase4(h_ref, out_ref, p):
        def do_chunk(glob, n):
            loc = glob - p * NP
            pltpu.sync_copy(agg.at[pl.ds(loc, n)], r0.at[pl.ds(0, n)])
            pltpu.sync_copy(h_ref.at[pl.ds(glob, n)], r1.at[pl.ds(0, n)])

            @plsc.parallel_loop(0, n, unroll=4)
            def _(i):
                nloc = jnp.full((16,), glob + i, jnp.int32)
                dn16 = plsc.load_gather(
                    ld, [lax.shift_right_logical(nloc, 7),
                         jnp.bitwise_and(nloc, 127)])
                keep = dn16 > 0.0
                for j in range(8):
                    sl = pl.ds(j * 16, 16)
                    r0[i, sl] = jnp.where(keep, r0[i, sl], r1[i, sl])

            pltpu.sync_copy(r0.at[pl.ds(0, n)], out_ref.at[pl.ds(glob, n)])

        base0 = p * NP

        @pl.when(s < 15)
        def _():
            @pl.loop(0, 4)
            def _(q):
                do_chunk(base0 + s * 320 + q * 80, 80)

        @pl.when(s == 15)
        def _():
            do_chunk(base0 + 4800, 80)
            do_chunk(base0 + 4880, 80)
            do_chunk(base0 + 4960, 40)

    for p in range(2):
        @pl.when(c == 0)
        def _():
            phase2(h0_ref, p)

        @pl.when(c == 1)
        def _():
            phase2(h1_ref, p)

        plsc.subcore_barrier()

        @pl.when(c == 0)
        def _():
            phase4(h0_ref, out0_ref, p)

        @pl.when(c == 1)
        def _():
            phase4(h1_ref, out1_ref, p)

        if p == 0:
            plsc.subcore_barrier()

            @pl.loop(0, 80)
            def _(k):
                z = jnp.zeros((16,), jnp.float32)
                for j in range(8):
                    r0[k, pl.ds(j * 16, 16)] = z

            zero_agg()
            plsc.subcore_barrier()


@functools.cache
def _sc_main_call():
    return pl.kernel(
        _sc_main_body,
        out_type=(
            jax.ShapeDtypeStruct((N, FH), jnp.float32),
            jax.ShapeDtypeStruct((N, FH), jnp.float32),
        ),
        mesh=plsc.VectorSubcoreMesh(core_axis_name="c", subcore_axis_name="s",
                                    num_cores=NC, num_subcores=NS),
        compiler_params=pltpu.CompilerParams(needs_layout_passes=False),
        scratch_types=[
            pltpu.VMEM((N,), jnp.float32),
            pltpu.VMEM((N,), jnp.float32),
            pltpu.VMEM((128,), jnp.float32),
            pltpu.VMEM((EPT,), jnp.int32),
            pltpu.VMEM((EPT,), jnp.int32),
            pltpu.VMEM((80, 128), jnp.float32),
            pltpu.VMEM((C2, FH), jnp.float32),
            pltpu.VMEM((C2, FH), jnp.float32),
            pltpu.VMEM((C2,), jnp.float32),
            pltpu.VMEM((C2,), jnp.int32),
            pltpu.VMEM((C2,), jnp.int32),
            pltpu.VMEM_SHARED((NP + 8, FH), jnp.float32),
            pltpu.SemaphoreType.DMA,
            pltpu.SemaphoreType.DMA,
            pltpu.SemaphoreType.DMA,
        ],
    )


@jax.jit
def kernel(x, edge_index, W, att_src, att_dst):
    h0, h1, as5, ad5, mz = _tc_call(x, W, att_src, att_dst)
    src = edge_index[0]
    dst = edge_index[1]
    att_state = jax.new_ref(jnp.zeros((N * N,), jnp.float32))
    den = _sc_att_call()(as5, ad5, mz, src, dst, att_state)
    out0, out1 = _sc_main_call()(as5, ad5, mz, src, dst, den, h0, h1)
    att = att_state[...].reshape(N, N)
    out = jnp.concatenate([out0, out1], axis=1)
    return out, att


# speedup vs baseline: 5.6224x; 1.0373x over previous
import functools

import jax
import jax.numpy as jnp
from jax import lax
from jax.experimental import pallas as pl
from jax.experimental.pallas import tpu as pltpu
from jax.experimental.pallas import tpu_sc as plsc

N = 10000
NP = N // 2
E = 160000
F = 256
FH = 128
NC = 2
NS = 16
RB = 2000
EPT = E // NS
C2 = 80
NCH = EPT // C2
E3 = E // 25


def _tc_body(x_ref, w_ref, as_ref, ad_ref, h0_ref, h1_ref,
             as5_ref, ad5_ref, m_ref, acc_ref):
    i = pl.program_id(0)
    h = lax.dot_general(x_ref[...], w_ref[...], (((1,), (1,)), ((), ())),
                        preferred_element_type=jnp.float32)
    h0_ref[...] = h[:, :FH]
    h1_ref[...] = h[:, FH:]
    a_s = lax.dot_general(as_ref[...], h, (((1,), (1,)), ((), ())),
                          preferred_element_type=jnp.float32)
    a_d = lax.dot_general(ad_ref[...], h, (((1,), (1,)), ((), ())),
                          preferred_element_type=jnp.float32)
    as5_ref[0, 0, :] = a_s[0]
    ad5_ref[0, 0, :] = a_d[0]

    @pl.when(i == 0)
    def _():
        acc_ref[0] = -jnp.inf
        acc_ref[1] = -jnp.inf

    acc_ref[0] = jnp.maximum(acc_ref[0], jnp.max(a_s))
    acc_ref[1] = jnp.maximum(acc_ref[1], jnp.max(a_d))

    @pl.when(i == pl.num_programs(0) - 1)
    def _():
        m_ref[...] = jnp.full((1, 128), acc_ref[0] + acc_ref[1], jnp.float32)


_tc_call = pl.pallas_call(
    _tc_body,
    grid=(N // RB,),
    in_specs=[
        pl.BlockSpec((RB, F), lambda i: (i, 0)),
        pl.BlockSpec((F, F), lambda i: (0, 0)),
        pl.BlockSpec((1, F), lambda i: (0, 0)),
        pl.BlockSpec((1, F), lambda i: (0, 0)),
    ],
    out_specs=[
        pl.BlockSpec((RB, FH), lambda i: (i, 0)),
        pl.BlockSpec((RB, FH), lambda i: (i, 0)),
        pl.BlockSpec((1, 1, RB), lambda i: (i, 0, 0)),
        pl.BlockSpec((1, 1, RB), lambda i: (i, 0, 0)),
        pl.BlockSpec((1, 128), lambda i: (0, 0)),
    ],
    out_shape=[
        jax.ShapeDtypeStruct((N, FH), jnp.float32),
        jax.ShapeDtypeStruct((N, FH), jnp.float32),
        jax.ShapeDtypeStruct((N // RB, 1, RB), jnp.float32),
        jax.ShapeDtypeStruct((N // RB, 1, RB), jnp.float32),
        jax.ShapeDtypeStruct((1, 128), jnp.float32),
    ],
    scratch_shapes=[pltpu.SMEM((2,), jnp.float32)],
)


def _lrelu(v):
    return jnp.where(v > 0, v, 0.2 * v)


def _sc_att_body(as5_ref, ad5_ref, m_ref, src_ref, dst_ref,
                 att_ref, den_ref,
                 asv, adv, mv, sv_, dv_, ld, s3v, d3v,
                 fb0, fb1, nv0, nv1, sden, sem):
    c = lax.axis_index("c")
    s = lax.axis_index("s")
    wid = c * NS + s
    iot = lax.iota(jnp.int32, 16)

    for i in range(N // RB):
        pltpu.sync_copy(as5_ref.at[i, 0], asv.at[pl.ds(i * RB, RB)])
        pltpu.sync_copy(ad5_ref.at[i, 0], adv.at[pl.ds(i * RB, RB)])
    pltpu.sync_copy(m_ref.at[0], mv)
    pltpu.sync_copy(src_ref.at[pl.ds(s * EPT, EPT)], sv_)
    pltpu.sync_copy(dst_ref.at[pl.ds(s * EPT, EPT)], dv_)

    @pl.loop(0, 80)
    def _(k):
        z = jnp.zeros((16,), jnp.float32)
        for j in range(8):
            ld[k, pl.ds(j * 16, 16)] = z

    @pl.when(s == 0)
    def _():
        pltpu.sync_copy(ld, sden)

    plsc.subcore_barrier()

    m_l = _lrelu(mv[pl.ds(0, 16)])

    def edge_group(svec, dvec):
        e = plsc.load_gather(asv, [svec]) + plsc.load_gather(adv, [dvec])
        return jnp.exp(_lrelu(e) - m_l)

    @plsc.parallel_loop(0, EPT // 16, unroll=2)
    def _(k):
        sl = pl.ds(k * 16, 16)
        dvec = dv_[sl]
        ex = edge_group(sv_[sl], dvec)
        plsc.addupdate_scatter(
            ld, [lax.shift_right_logical(dvec, 7), jnp.bitwise_and(dvec, 127)],
            ex)

    for g in range(5):
        pltpu.sync_copy(ld.at[pl.ds(g * 16, 16)], sden.at[iot + g * 16],
                        add=True)
    plsc.subcore_barrier()
    pltpu.sync_copy(sden, ld)

    @pl.when(jnp.logical_and(c == 0, s == 0))
    def _():
        pltpu.sync_copy(ld, den_ref)

    def denom_group(dvec):
        return plsc.load_gather(
            ld, [lax.shift_right_logical(dvec, 7), jnp.bitwise_and(dvec, 127)])

    @pl.when(wid < 25)
    def _():
        pltpu.sync_copy(src_ref.at[pl.ds(wid * E3, E3)], s3v)
        pltpu.sync_copy(dst_ref.at[pl.ds(wid * E3, E3)], d3v)

        @pl.loop(0, E3 // 256)
        def _(jj):
            descs = []
            for half, (fb, nv) in enumerate(((fb0, nv0), (fb1, nv1))):
                j = jj * 2 + half
                for g in range(8):
                    sl = pl.ds(j * 128 + g * 16, 16)
                    svec = s3v[sl]
                    dvec = d3v[sl]
                    ex = edge_group(svec, dvec)
                    nv[pl.ds(g * 16, 16)] = ex / denom_group(dvec)
                    fb[pl.ds(g * 16, 16)] = svec * N + dvec
                descs.append(pltpu.async_copy(nv, att_ref.at[fb], sem))
            for d in descs:
                d.wait()


@functools.cache
def _sc_att_call():
    return pl.kernel(
        _sc_att_body,
        out_type=jax.ShapeDtypeStruct((80, 128), jnp.float32),
        mesh=plsc.VectorSubcoreMesh(core_axis_name="c", subcore_axis_name="s",
                                    num_cores=NC, num_subcores=NS),
        compiler_params=pltpu.CompilerParams(needs_layout_passes=False),
        scratch_types=[
            pltpu.VMEM((N,), jnp.float32),
            pltpu.VMEM((N,), jnp.float32),
            pltpu.VMEM((128,), jnp.float32),
            pltpu.VMEM((EPT,), jnp.int32),
            pltpu.VMEM((EPT,), jnp.int32),
            pltpu.VMEM((80, 128), jnp.float32),
            pltpu.VMEM((E3,), jnp.int32),
            pltpu.VMEM((E3,), jnp.int32),
            pltpu.VMEM((128,), jnp.int32),
            pltpu.VMEM((128,), jnp.int32),
            pltpu.VMEM((128,), jnp.float32),
            pltpu.VMEM((128,), jnp.float32),
            pltpu.VMEM_SHARED((80, 128), jnp.float32),
            pltpu.SemaphoreType.DMA,
        ],
    )


def _sc_main_body(as5_ref, ad5_ref, m_ref, src_ref, dst_ref, den_ref,
                  h0_ref, h1_ref, out0_ref, out1_ref,
                  asv, adv, mv, sv_, dv_, ld, r0, r1, r2, nb,
                  dl0, dl1, dl2, agg, sem, sem_a, sem_b, sem_c):
    c = lax.axis_index("c")
    s = lax.axis_index("s")

    for i in range(N // RB):
        pltpu.sync_copy(as5_ref.at[i, 0], asv.at[pl.ds(i * RB, RB)])
        pltpu.sync_copy(ad5_ref.at[i, 0], adv.at[pl.ds(i * RB, RB)])
    pltpu.sync_copy(m_ref.at[0], mv)
    pltpu.sync_copy(src_ref.at[pl.ds(s * EPT, EPT)], sv_)
    pltpu.sync_copy(dst_ref.at[pl.ds(s * EPT, EPT)], dv_)
    pltpu.sync_copy(den_ref, ld)

    @pl.loop(0, 80)
    def _(k):
        z = jnp.zeros((16,), jnp.float32)
        for j in range(8):
            r0[k, pl.ds(j * 16, 16)] = z

    def zero_agg():
        @pl.when(s < 15)
        def _():
            @pl.loop(0, 4)
            def _(q):
                pltpu.sync_copy(r0, agg.at[pl.ds(s * 320 + q * 80, 80)])

        @pl.when(s == 15)
        def _():
            pltpu.sync_copy(r0, agg.at[pl.ds(4800, 80)])
            pltpu.sync_copy(r0, agg.at[pl.ds(4880, 80)])
            pltpu.sync_copy(r0.at[pl.ds(0, 48)], agg.at[pl.ds(4960, 48)])

    zero_agg()
    plsc.subcore_barrier()

    m_l = _lrelu(mv[pl.ds(0, 16)])

    def edge_group(svec, dvec):
        e = plsc.load_gather(asv, [svec]) + plsc.load_gather(adv, [dvec])
        return jnp.exp(_lrelu(e) - m_l)

    def denom_group(dvec):
        return plsc.load_gather(
            ld, [lax.shift_right_logical(dvec, 7), jnp.bitwise_and(dvec, 127)])

    def phase2(h_ref, p):
        def gissue(k, buf, gsem):
            return pltpu.async_copy(h_ref.at[sv_.at[pl.ds(k * C2, C2)]], buf,
                                    gsem)

        def process(k, buf, desc, dlb):
            for g in range(5):
                sl = pl.ds(k * C2 + g * 16, 16)
                dvec = dv_[sl]
                ex = edge_group(sv_[sl], dvec)
                nb[pl.ds(g * 16, 16)] = ex / denom_group(dvec)
                dloc = dvec - p * NP
                ok = jnp.logical_and(dloc >= 0, dloc < NP)
                dlb[pl.ds(g * 16, 16)] = jnp.where(ok, dloc, NP)
            desc.wait()

            @plsc.parallel_loop(0, C2, unroll=4)
            def _(i):
                w = plsc.load_gather(nb, [jnp.full((16,), i, jnp.int32)])
                for j in range(8):
                    sl = pl.ds(j * 16, 16)
                    buf[i, sl] = buf[i, sl] * w

            return pltpu.async_copy(buf, agg.at[dlb], sem, add=True)

        @pl.loop(0, NCH // 3)
        def _(kk):
            k = kk * 3
            da = gissue(k, r0, sem_a)
            db = gissue(k + 1, r1, sem_b)
            dc = gissue(k + 2, r2, sem_c)
            wa = process(k, r0, da, dl0)
            wb = process(k + 1, r1, db, dl1)
            wc = process(k + 2, r2, dc, dl2)
            wa.wait()
            wb.wait()
            wc.wait()

        k_last = NCH - 2
        da = gissue(k_last, r0, sem_a)
        db = gissue(k_last + 1, r1, sem_b)
        process(k_last, r0, da, dl0).wait()
        process(k_last + 1, r1, db, dl1).wait()

    def phase4(h_ref, out_ref, p):
        def do_chunk(glob, n):
            loc = glob - p * NP
            pltpu.sync_copy(agg.at[pl.ds(loc, n)], r0.at[pl.ds(0, n)])
            pltpu.sync_copy(h_ref.at[pl.ds(glob, n)], r1.at[pl.ds(0, n)])

            @plsc.parallel_loop(0, n, unroll=4)
            def _(i):
                nloc = jnp.full((16,), glob + i, jnp.int32)
                dn16 = plsc.load_gather(
                    ld, [lax.shift_right_logical(nloc, 7),
                         jnp.bitwise_and(nloc, 127)])
                keep = dn16 > 0.0
                for j in range(8):
                    sl = pl.ds(j * 16, 16)
                    r0[i, sl] = jnp.where(keep, r0[i, sl], r1[i, sl])

            pltpu.sync_copy(r0.at[pl.ds(0, n)], out_ref.at[pl.ds(glob, n)])

        base0 = p * NP

        @pl.when(s < 15)
        def _():
            @pl.loop(0, 4)
            def _(q):
                do_chunk(base0 + s * 320 + q * 80, 80)

        @pl.when(s == 15)
        def _():
            do_chunk(base0 + 4800, 80)
            do_chunk(base0 + 4880, 80)
            do_chunk(base0 + 4960, 40)

    for p in range(2):
        @pl.when(c == 0)
        def _():
            phase2(h0_ref, p)

        @pl.when(c == 1)
        def _():
            phase2(h1_ref, p)

        plsc.subcore_barrier()

        @pl.when(c == 0)
        def _():
            phase4(h0_ref, out0_ref, p)

        @pl.when(c == 1)
        def _():
            phase4(h1_ref, out1_ref, p)

        if p == 0:
            plsc.subcore_barrier()

            @pl.loop(0, 80)
            def _(k):
                z = jnp.zeros((16,), jnp.float32)
                for j in range(8):
                    r0[k, pl.ds(j * 16, 16)] = z

            zero_agg()
            plsc.subcore_barrier()


@functools.cache
def _sc_main_call():
    return pl.kernel(
        _sc_main_body,
        out_type=(
            jax.ShapeDtypeStruct((N, FH), jnp.float32),
            jax.ShapeDtypeStruct((N, FH), jnp.float32),
        ),
        mesh=plsc.VectorSubcoreMesh(core_axis_name="c", subcore_axis_name="s",
                                    num_cores=NC, num_subcores=NS),
        compiler_params=pltpu.CompilerParams(needs_layout_passes=False),
        scratch_types=[
            pltpu.VMEM((N,), jnp.float32),
            pltpu.VMEM((N,), jnp.float32),
            pltpu.VMEM((128,), jnp.float32),
            pltpu.VMEM((EPT,), jnp.int32),
            pltpu.VMEM((EPT,), jnp.int32),
            pltpu.VMEM((80, 128), jnp.float32),
            pltpu.VMEM((C2, FH), jnp.float32),
            pltpu.VMEM((C2, FH), jnp.float32),
            pltpu.VMEM((C2, FH), jnp.float32),
            pltpu.VMEM((C2,), jnp.float32),
            pltpu.VMEM((C2,), jnp.int32),
            pltpu.VMEM((C2,), jnp.int32),
            pltpu.VMEM((C2,), jnp.int32),
            pltpu.VMEM_SHARED((NP + 8, FH), jnp.float32),
            pltpu.SemaphoreType.DMA,
            pltpu.SemaphoreType.DMA,
            pltpu.SemaphoreType.DMA,
            pltpu.SemaphoreType.DMA,
        ],
    )


@jax.jit
def kernel(x, edge_index, W, att_src, att_dst):
    h0, h1, as5, ad5, mz = _tc_call(x, W, att_src, att_dst)
    src = edge_index[0]
    dst = edge_index[1]
    att_state = jax.new_ref(jnp.zeros((N * N,), jnp.float32))
    den = _sc_att_call()(as5, ad5, mz, src, dst, att_state)
    out0, out1 = _sc_main_call()(as5, ad5, mz, src, dst, den, h0, h1)
    att = att_state[...].reshape(N, N)
    out = jnp.concatenate([out0, out1], axis=1)
    return out, att
